# baseline (device time: 142822 ns/iter reference)
import functools

import jax
import jax.numpy as jnp
from jax import lax
from jax.experimental import pallas as pl
from jax.experimental.pallas import tpu as pltpu

N_DEV = 32
E_PER = 4
N_TOK = 2048
D = 512
H = 1024
N_PLANE = 8
N_Z = 4
STRIP = N_TOK // N_PLANE
SUB = STRIP // N_Z


def _moe_body(x_ref, ridx_ref, w_ref, out_ref, obuf, wbuf, pstage, zstage,
              p_rs_send, p_rs_recv, z_rs_send, z_rs_recv,
              z_ag_send, z_ag_recv, p_ag_send, p_ag_recv):
    my = lax.axis_index("i")
    zz = lax.div(my, N_PLANE)
    q = lax.rem(my, N_PLANE)
    plane_next = zz * N_PLANE + lax.rem(q + 1, N_PLANE)
    plane_prev = zz * N_PLANE + lax.rem(q + N_PLANE - 1, N_PLANE)
    z_next = lax.rem(my + N_PLANE, N_DEV)
    z_prev = lax.rem(my + N_DEV - N_PLANE, N_DEV)

    barrier = pltpu.get_barrier_semaphore()
    for nbr in (plane_prev, plane_next, z_prev, z_next):
        pl.semaphore_signal(barrier, inc=1, device_id=(nbr,),
                            device_id_type=pl.DeviceIdType.MESH)
    pl.semaphore_wait(barrier, 4)

    for k in range(E_PER):
        wbuf[k, :, :] = w_ref[k].astype(jnp.bfloat16)

    def compute_strip(c):
        sl = pl.ds(c * STRIP, STRIP)
        xc = x_ref[sl, :].astype(jnp.bfloat16)
        rc = ridx_ref[sl, :]
        acc = jnp.zeros((STRIP, H), jnp.float32)
        for k in range(E_PER):
            mask = (rc == E_PER * my + k).astype(jnp.bfloat16)
            acc = acc + jnp.dot(xc * mask, wbuf[k],
                                preferred_element_type=jnp.float32)
        out_ref[sl, :] = acc
        obuf[sl, :] = acc.astype(jnp.bfloat16)

    ops = []

    def remote_copy(src_sl, dst_ref, send_sems, recv_sems, h, target):
        op = pltpu.make_async_remote_copy(
            src_ref=obuf.at[src_sl],
            dst_ref=dst_ref,
            send_sem=send_sems.at[h],
            recv_sem=recv_sems.at[h],
            device_id=(target,),
            device_id_type=pl.DeviceIdType.MESH,
        )
        op.start()
        ops.append(op)
        return op

    compute_strip(q)
    for h in range(N_PLANE - 1):
        s = lax.rem(q - h + 2 * N_PLANE, N_PLANE)
        op = remote_copy(pl.ds(s * STRIP, STRIP), pstage.at[h],
                         p_rs_send, p_rs_recv, h, plane_next)
        r = lax.rem(q - h - 1 + 2 * N_PLANE, N_PLANE)
        compute_strip(r)
        op.wait_recv()
        sl = pl.ds(r * STRIP, STRIP)
        acc = out_ref[sl, :] + pstage[h].astype(jnp.float32)
        out_ref[sl, :] = acc
        obuf[sl, :] = acc.astype(jnp.bfloat16)
    S = lax.rem(q + 1, N_PLANE)

    for h in range(N_Z - 1):
        s = lax.rem(zz - h + 2 * N_Z, N_Z)
        op = remote_copy(pl.ds(S * STRIP + s * SUB, SUB), zstage.at[h],
                         z_rs_send, z_rs_recv, h, z_next)
        op.wait_recv()
        r = lax.rem(zz - h - 1 + 2 * N_Z, N_Z)
        sl = pl.ds(S * STRIP + r * SUB, SUB)
        acc = out_ref[sl, :] + zstage[h].astype(jnp.float32)
        out_ref[sl, :] = acc
        obuf[sl, :] = acc.astype(jnp.bfloat16)

    for h in range(N_Z - 1):
        s = lax.rem(zz + 1 - h + 2 * N_Z, N_Z)
        sl = pl.ds(S * STRIP + s * SUB, SUB)
        op = remote_copy(sl, obuf.at[sl], z_ag_send, z_ag_recv, h, z_next)
        op.wait_recv()

    for h in range(N_PLANE - 1):
        s = lax.rem(q + 1 - h + 2 * N_PLANE, N_PLANE)
        sl = pl.ds(s * STRIP, STRIP)
        op = remote_copy(sl, obuf.at[sl], p_ag_send, p_ag_recv, h,
                         plane_next)
        op.wait_recv()

    out_ref[:, :] = obuf[:, :].astype(jnp.float32)

    for op in ops:
        op.wait_send()

    @functools.partial(pl.run_scoped, sem=pltpu.SemaphoreType.REGULAR)
    def _(sem):
        for nbr in (plane_prev, plane_next, z_prev, z_next):
            pl.semaphore_signal(sem, inc=1, device_id=(nbr,),
                                device_id_type=pl.DeviceIdType.MESH)
        pl.semaphore_wait(sem, 4)


def kernel(x, router_W, route_idx, expert_W):
    del router_W
    np1 = N_PLANE - 1
    nz1 = N_Z - 1
    dma = pltpu.SemaphoreType.DMA
    return pl.pallas_call(
        _moe_body,
        out_shape=jax.ShapeDtypeStruct((N_TOK, H), jnp.float32),
        in_specs=[
            pl.BlockSpec(memory_space=pltpu.VMEM),
            pl.BlockSpec(memory_space=pltpu.VMEM),
            pl.BlockSpec(memory_space=pltpu.VMEM),
        ],
        out_specs=pl.BlockSpec(memory_space=pltpu.VMEM),
        scratch_shapes=[
            pltpu.VMEM((N_TOK, H), jnp.bfloat16),
            pltpu.VMEM((E_PER, D, H), jnp.bfloat16),
            pltpu.VMEM((np1, STRIP, H), jnp.bfloat16),
            pltpu.VMEM((nz1, SUB, H), jnp.bfloat16),
            dma((np1,)), dma((np1,)),
            dma((nz1,)), dma((nz1,)),
            dma((nz1,)), dma((nz1,)),
            dma((np1,)), dma((np1,)),
        ],
        compiler_params=pltpu.CompilerParams(collective_id=0),
    )(x, route_idx.astype(jnp.int32), expert_W)


# device time: 141814 ns/iter; 1.0071x vs baseline; 1.0071x over previous
import functools

import jax
import jax.numpy as jnp
from jax import lax
from jax.experimental import pallas as pl
from jax.experimental.pallas import tpu as pltpu

N_DEV = 32
E_PER = 4
N_TOK = 2048
D = 512
H = 1024
N_PLANE = 8
N_Z = 4
STRIP = N_TOK // N_PLANE
SUB = STRIP // N_Z


def _moe_body(x_ref, ridx_ref, w_ref, out_ref, obuf, wbuf, pstage, zstage,
              p_rs_send, p_rs_recv, z_rs_send, z_rs_recv,
              z_ag_send, z_ag_recv, p_ag_send, p_ag_recv):
    my = lax.axis_index("i")
    zz = lax.div(my, N_PLANE)
    q = lax.rem(my, N_PLANE)
    plane_next = zz * N_PLANE + lax.rem(q + 1, N_PLANE)
    plane_prev = zz * N_PLANE + lax.rem(q + N_PLANE - 1, N_PLANE)
    z_next = lax.rem(my + N_PLANE, N_DEV)
    z_prev = lax.rem(my + N_DEV - N_PLANE, N_DEV)

    barrier = pltpu.get_barrier_semaphore()
    for nbr in (plane_prev, plane_next, z_prev, z_next):
        pl.semaphore_signal(barrier, inc=1, device_id=(nbr,),
                            device_id_type=pl.DeviceIdType.MESH)
    pl.semaphore_wait(barrier, 4)

    for k in range(E_PER):
        wbuf[k, :, :] = w_ref[k].astype(jnp.bfloat16)

    def compute_strip(c):
        sl = pl.ds(c * STRIP, STRIP)
        xc = x_ref[sl, :].astype(jnp.bfloat16)
        rc = ridx_ref[sl, :]
        acc = jnp.zeros((STRIP, H), jnp.float32)
        for k in range(E_PER):
            mask = (rc == E_PER * my + k).astype(jnp.bfloat16)
            acc = acc + jnp.dot(xc * mask, wbuf[k],
                                preferred_element_type=jnp.float32)
        obuf[sl, :] = acc.astype(jnp.bfloat16)

    ops = []

    def remote_copy(src_sl, dst_ref, send_sems, recv_sems, h, target):
        op = pltpu.make_async_remote_copy(
            src_ref=obuf.at[src_sl],
            dst_ref=dst_ref,
            send_sem=send_sems.at[h],
            recv_sem=recv_sems.at[h],
            device_id=(target,),
            device_id_type=pl.DeviceIdType.MESH,
        )
        op.start()
        ops.append(op)
        return op

    compute_strip(q)
    for h in range(N_PLANE - 1):
        s = lax.rem(q - h + 2 * N_PLANE, N_PLANE)
        op = remote_copy(pl.ds(s * STRIP, STRIP), pstage.at[h],
                         p_rs_send, p_rs_recv, h, plane_next)
        r = lax.rem(q - h - 1 + 2 * N_PLANE, N_PLANE)
        compute_strip(r)
        op.wait_recv()
        sl = pl.ds(r * STRIP, STRIP)
        obuf[sl, :] = obuf[sl, :] + pstage[h]
    S = lax.rem(q + 1, N_PLANE)

    for h in range(N_Z - 1):
        s = lax.rem(zz - h + 2 * N_Z, N_Z)
        op = remote_copy(pl.ds(S * STRIP + s * SUB, SUB), zstage.at[h],
                         z_rs_send, z_rs_recv, h, z_next)
        op.wait_recv()
        r = lax.rem(zz - h - 1 + 2 * N_Z, N_Z)
        sl = pl.ds(S * STRIP + r * SUB, SUB)
        obuf[sl, :] = obuf[sl, :] + zstage[h]

    for h in range(N_Z - 1):
        s = lax.rem(zz + 1 - h + 2 * N_Z, N_Z)
        sl = pl.ds(S * STRIP + s * SUB, SUB)
        op = remote_copy(sl, obuf.at[sl], z_ag_send, z_ag_recv, h, z_next)
        op.wait_recv()

    for h in range(N_PLANE - 1):
        s = lax.rem(q + 1 - h + 2 * N_PLANE, N_PLANE)
        sl = pl.ds(s * STRIP, STRIP)
        op = remote_copy(sl, obuf.at[sl], p_ag_send, p_ag_recv, h,
                         plane_next)
        op.wait_recv()

    out_ref[:, :] = obuf[:, :].astype(jnp.float32)

    for op in ops:
        op.wait_send()

    @functools.partial(pl.run_scoped, sem=pltpu.SemaphoreType.REGULAR)
    def _(sem):
        for nbr in (plane_prev, plane_next, z_prev, z_next):
            pl.semaphore_signal(sem, inc=1, device_id=(nbr,),
                                device_id_type=pl.DeviceIdType.MESH)
        pl.semaphore_wait(sem, 4)


def kernel(x, router_W, route_idx, expert_W):
    del router_W
    np1 = N_PLANE - 1
    nz1 = N_Z - 1
    dma = pltpu.SemaphoreType.DMA
    return pl.pallas_call(
        _moe_body,
        out_shape=jax.ShapeDtypeStruct((N_TOK, H), jnp.float32),
        in_specs=[
            pl.BlockSpec(memory_space=pltpu.VMEM),
            pl.BlockSpec(memory_space=pltpu.VMEM),
            pl.BlockSpec(memory_space=pltpu.VMEM),
        ],
        out_specs=pl.BlockSpec(memory_space=pltpu.VMEM),
        scratch_shapes=[
            pltpu.VMEM((N_TOK, H), jnp.bfloat16),
            pltpu.VMEM((E_PER, D, H), jnp.bfloat16),
            pltpu.VMEM((np1, STRIP, H), jnp.bfloat16),
            pltpu.VMEM((nz1, SUB, H), jnp.bfloat16),
            dma((np1,)), dma((np1,)),
            dma((nz1,)), dma((nz1,)),
            dma((nz1,)), dma((nz1,)),
            dma((np1,)), dma((np1,)),
        ],
        compiler_params=pltpu.CompilerParams(collective_id=0),
    )(x, route_idx.astype(jnp.int32), expert_W)


# device time: 131201 ns/iter; 1.0886x vs baseline; 1.0809x over previous
import functools

import jax
import jax.numpy as jnp
from jax import lax
from jax.experimental import pallas as pl
from jax.experimental.pallas import tpu as pltpu

N_DEV = 32
E_PER = 4
N_TOK = 2048
D = 512
H = 1024
N_PLANE = 8
N_Z = 4
N_Y = 4
HALF = N_TOK // 2
STRIP = HALF // N_Y
SUB = STRIP // N_Z


def _moe_body(x_ref, ridx_ref, w_ref, out_ref, obuf, wbuf,
              xstage, ystage, zstage,
              x_rs_send, x_rs_recv, y_rs_send, y_rs_recv,
              z_rs_send, z_rs_recv, z_ag_send, z_ag_recv,
              y_ag_send, y_ag_recv, x_ag_send, x_ag_recv):
    my = lax.axis_index("i")
    zz = lax.div(my, N_PLANE)
    q = lax.rem(my, N_PLANE)
    xc = lax.rem(lax.div(q + 1, 2), 2)
    yy = lax.div(q, 2)
    parity = lax.rem(q, 2)
    x_partner = zz * N_PLANE + jnp.bitwise_xor(q, 1)
    y_next = zz * N_PLANE + lax.rem(q + jnp.where(parity == 0, 3, 1), N_PLANE)
    y_prev = zz * N_PLANE + lax.rem(q + jnp.where(parity == 0, 7, 5), N_PLANE)
    zdev = [lax.rem(my + (1 + j) * N_PLANE, N_DEV) for j in range(N_Z - 1)]

    barrier = pltpu.get_barrier_semaphore()
    for nbr in (x_partner, y_next, y_prev, *zdev):
        pl.semaphore_signal(barrier, inc=1, device_id=(nbr,),
                            device_id_type=pl.DeviceIdType.MESH)
    pl.semaphore_wait(barrier, 6)

    for k in range(E_PER):
        wbuf[k, :, :] = w_ref[k].astype(jnp.bfloat16)

    def compute_strip(c):
        sl = pl.ds(c * STRIP, STRIP)
        xrows = x_ref[sl, :].astype(jnp.bfloat16)
        rc = ridx_ref[sl, :]
        acc = jnp.zeros((STRIP, H), jnp.float32)
        for k in range(E_PER):
            mask = (rc == E_PER * my + k).astype(jnp.bfloat16)
            acc = acc + jnp.dot(xrows * mask, wbuf[k],
                                preferred_element_type=jnp.float32)
        obuf[sl, :] = acc.astype(jnp.bfloat16)

    ops = []

    def remote_copy(src_sl, dst_ref, send_sem, recv_sem, target):
        op = pltpu.make_async_remote_copy(
            src_ref=obuf.at[src_sl],
            dst_ref=dst_ref,
            send_sem=send_sem,
            recv_sem=recv_sem,
            device_id=(target,),
            device_id_type=pl.DeviceIdType.MESH,
        )
        op.start()
        ops.append(op)
        return op

    mybase = xc * HALF
    theirbase = (1 - xc) * HALF

    for j in range(N_Y):
        compute_strip((1 - xc) * N_Y + j)
    xop = remote_copy(pl.ds(theirbase, HALF), xstage,
                      x_rs_send.at[0], x_rs_recv.at[0], x_partner)
    for j in range(N_Y):
        compute_strip(xc * N_Y + j)
    xop.wait_recv()
    sl = pl.ds(mybase, HALF)
    obuf[sl, :] = obuf[sl, :] + xstage[:, :]

    def strip_rows(s):
        return pl.ds(mybase + s * STRIP, STRIP)

    for h in range(N_Y - 1):
        s = lax.rem(yy - h + 2 * N_Y, N_Y)
        op = remote_copy(strip_rows(s), ystage.at[h],
                         y_rs_send.at[h], y_rs_recv.at[h], y_next)
        op.wait_recv()
        r = lax.rem(yy - h - 1 + 2 * N_Y, N_Y)
        sl = strip_rows(r)
        obuf[sl, :] = obuf[sl, :] + ystage[h]
    S = lax.rem(yy + 1, N_Y)

    def sub_rows(s):
        return pl.ds(mybase + S * STRIP + s * SUB, SUB)

    def slot_on(t):
        return jnp.where(zz > t, zz - 1, zz)

    for j in range(N_Z - 1):
        t = lax.rem(zz + 1 + j, N_Z)
        sl_t = slot_on(t)
        remote_copy(sub_rows(t), zstage.at[sl_t],
                    z_rs_send.at[j], z_rs_recv.at[sl_t], zdev[j])
    for i in range(N_Z - 1):
        rcv = pltpu.make_async_remote_copy(
            src_ref=zstage.at[i], dst_ref=zstage.at[i],
            send_sem=z_rs_send.at[i], recv_sem=z_rs_recv.at[i],
            device_id=(zdev[0],), device_id_type=pl.DeviceIdType.MESH,
        )
        rcv.wait_recv()
    sl = sub_rows(zz)
    obuf[sl, :] = obuf[sl, :] + zstage[0] + zstage[1] + zstage[2]

    for j in range(N_Z - 1):
        t = lax.rem(zz + 1 + j, N_Z)
        remote_copy(sub_rows(zz), obuf.at[sub_rows(zz)],
                    z_ag_send.at[j], z_ag_recv.at[slot_on(t)], zdev[j])
    for i in range(N_Z - 1):
        s_i = jnp.where(i >= zz, i + 1, i)
        rcv = pltpu.make_async_remote_copy(
            src_ref=obuf.at[sub_rows(s_i)], dst_ref=obuf.at[sub_rows(s_i)],
            send_sem=z_ag_send.at[i], recv_sem=z_ag_recv.at[i],
            device_id=(zdev[0],), device_id_type=pl.DeviceIdType.MESH,
        )
        rcv.wait_recv()

    for h in range(N_Y - 1):
        s = lax.rem(yy + 1 - h + 2 * N_Y, N_Y)
        sl = strip_rows(s)
        op = remote_copy(sl, obuf.at[sl],
                         y_ag_send.at[h], y_ag_recv.at[h], y_next)
        op.wait_recv()

    sl = pl.ds(mybase, HALF)
    xop = remote_copy(sl, obuf.at[sl],
                      x_ag_send.at[0], x_ag_recv.at[0], x_partner)
    xop.wait_recv()

    out_ref[:, :] = obuf[:, :].astype(jnp.float32)

    for op in ops:
        op.wait_send()

    @functools.partial(pl.run_scoped, sem=pltpu.SemaphoreType.REGULAR)
    def _(sem):
        for nbr in (x_partner, y_next, y_prev, *zdev):
            pl.semaphore_signal(sem, inc=1, device_id=(nbr,),
                                device_id_type=pl.DeviceIdType.MESH)
        pl.semaphore_wait(sem, 6)


def kernel(x, router_W, route_idx, expert_W):
    del router_W
    ny1 = N_Y - 1
    nz1 = N_Z - 1
    dma = pltpu.SemaphoreType.DMA
    return pl.pallas_call(
        _moe_body,
        out_shape=jax.ShapeDtypeStruct((N_TOK, H), jnp.float32),
        in_specs=[
            pl.BlockSpec(memory_space=pltpu.VMEM),
            pl.BlockSpec(memory_space=pltpu.VMEM),
            pl.BlockSpec(memory_space=pltpu.VMEM),
        ],
        out_specs=pl.BlockSpec(memory_space=pltpu.VMEM),
        scratch_shapes=[
            pltpu.VMEM((N_TOK, H), jnp.bfloat16),
            pltpu.VMEM((E_PER, D, H), jnp.bfloat16),
            pltpu.VMEM((HALF, H), jnp.bfloat16),
            pltpu.VMEM((ny1, STRIP, H), jnp.bfloat16),
            pltpu.VMEM((nz1, SUB, H), jnp.bfloat16),
            dma((1,)), dma((1,)),
            dma((ny1,)), dma((ny1,)),
            dma((nz1,)), dma((nz1,)),
            dma((nz1,)), dma((nz1,)),
            dma((ny1,)), dma((ny1,)),
            dma((1,)), dma((1,)),
        ],
        compiler_params=pltpu.CompilerParams(collective_id=0),
    )(x, route_idx.astype(jnp.int32), expert_W)


# device time: 94040 ns/iter; 1.5187x vs baseline; 1.3952x over previous
import functools

import jax
import jax.numpy as jnp
from jax import lax
from jax.experimental import pallas as pl
from jax.experimental.pallas import tpu as pltpu

N_DEV = 32
E_PER = 4
N_TOK = 2048
D = 512
H = 1024
N_PLANE = 8
N_Z = 4
N_Y = 4
HALF = N_TOK // 2
STRIP = HALF // N_Y
SUB = STRIP // N_Z


def _moe_body(x_ref, ridx_ref, w_ref, out_ref, obuf, wbuf,
              xstage, ystage, zstage,
              x_rs_send, x_rs_recv, y_rs_send, y_rs_recv,
              z_rs_send, z_rs_recv, z_ag_send, z_ag_recv,
              y_ag_send, y_ag_recv, x_ag_send, x_ag_recv):
    my = lax.axis_index("i")
    zz = lax.div(my, N_PLANE)
    q = lax.rem(my, N_PLANE)
    xc = lax.rem(lax.div(q + 1, 2), 2)
    yy = lax.div(q, 2)
    parity = lax.rem(q, 2)
    x_partner = zz * N_PLANE + jnp.bitwise_xor(q, 1)
    y_next = zz * N_PLANE + lax.rem(q + jnp.where(parity == 0, 3, 1), N_PLANE)
    y_prev = zz * N_PLANE + lax.rem(q + jnp.where(parity == 0, 7, 5), N_PLANE)
    zdev = [lax.rem(my + (1 + j) * N_PLANE, N_DEV) for j in range(N_Z - 1)]

    barrier = pltpu.get_barrier_semaphore()
    for nbr in (x_partner, y_next, y_prev, *zdev):
        pl.semaphore_signal(barrier, inc=1, device_id=(nbr,),
                            device_id_type=pl.DeviceIdType.MESH)
    pl.semaphore_wait(barrier, 6)

    for k in range(E_PER):
        wbuf[k, :, :] = w_ref[k].astype(jnp.bfloat16)

    def compute_strip(c):
        sl = pl.ds(c * STRIP, STRIP)
        xrows = x_ref[sl, :].astype(jnp.bfloat16)
        rc = ridx_ref[sl, :]
        acc = jnp.zeros((STRIP, H), jnp.float32)
        for k in range(E_PER):
            mask = (rc == E_PER * my + k).astype(jnp.bfloat16)
            acc = acc + jnp.dot(xrows * mask, wbuf[k],
                                preferred_element_type=jnp.float32)
        obuf[sl, :] = acc.astype(jnp.bfloat16)

    ops = []

    def remote_copy(src_sl, dst_ref, send_sem, recv_sem, target):
        op = pltpu.make_async_remote_copy(
            src_ref=obuf.at[src_sl],
            dst_ref=dst_ref,
            send_sem=send_sem,
            recv_sem=recv_sem,
            device_id=(target,),
            device_id_type=pl.DeviceIdType.MESH,
        )
        op.start()
        ops.append(op)
        return op

    mybase = xc * HALF
    theirbase = (1 - xc) * HALF

    def strip_rows(s):
        return pl.ds(mybase + s * STRIP, STRIP)

    for j in range(N_Y):
        s = lax.rem(yy - j + 2 * N_Y, N_Y)
        compute_strip((1 - xc) * N_Y + s)
        remote_copy(pl.ds(theirbase + s * STRIP, STRIP),
                    xstage.at[pl.ds(s * STRIP, STRIP)],
                    x_rs_send.at[j], x_rs_recv.at[j], x_partner)

    yops = []
    for j in range(N_Y):
        s = lax.rem(yy - j + 2 * N_Y, N_Y)
        compute_strip(xc * N_Y + s)
        p1 = pltpu.make_async_remote_copy(
            src_ref=xstage.at[pl.ds(0, STRIP)],
            dst_ref=xstage.at[pl.ds(0, STRIP)],
            send_sem=x_rs_send.at[j], recv_sem=x_rs_recv.at[j],
            device_id=(x_partner,), device_id_type=pl.DeviceIdType.MESH,
        )
        p1.wait_recv()
        sl = strip_rows(s)
        obuf[sl, :] = obuf[sl, :] + xstage[pl.ds(s * STRIP, STRIP)]
        if j >= 1:
            yops[j - 1].wait_recv()
            obuf[sl, :] = obuf[sl, :] + ystage[j - 1]
        if j < N_Y - 1:
            yops.append(remote_copy(sl, ystage.at[j],
                                    y_rs_send.at[j], y_rs_recv.at[j],
                                    y_next))
    S = lax.rem(yy + 1, N_Y)

    def sub_rows(s):
        return pl.ds(mybase + S * STRIP + s * SUB, SUB)

    def slot_on(t):
        return jnp.where(zz > t, zz - 1, zz)

    for j in range(N_Z - 1):
        t = lax.rem(zz + 1 + j, N_Z)
        sl_t = slot_on(t)
        remote_copy(sub_rows(t), zstage.at[sl_t],
                    z_rs_send.at[j], z_rs_recv.at[sl_t], zdev[j])
    for i in range(N_Z - 1):
        rcv = pltpu.make_async_remote_copy(
            src_ref=zstage.at[i], dst_ref=zstage.at[i],
            send_sem=z_rs_send.at[i], recv_sem=z_rs_recv.at[i],
            device_id=(zdev[0],), device_id_type=pl.DeviceIdType.MESH,
        )
        rcv.wait_recv()
    sl = sub_rows(zz)
    obuf[sl, :] = obuf[sl, :] + zstage[0] + zstage[1] + zstage[2]

    for j in range(N_Z - 1):
        t = lax.rem(zz + 1 + j, N_Z)
        remote_copy(sub_rows(zz), obuf.at[sub_rows(zz)],
                    z_ag_send.at[j], z_ag_recv.at[slot_on(t)], zdev[j])
    for i in range(N_Z - 1):
        s_i = jnp.where(i >= zz, i + 1, i)
        rcv = pltpu.make_async_remote_copy(
            src_ref=obuf.at[sub_rows(s_i)], dst_ref=obuf.at[sub_rows(s_i)],
            send_sem=z_ag_send.at[i], recv_sem=z_ag_recv.at[i],
            device_id=(zdev[0],), device_id_type=pl.DeviceIdType.MESH,
        )
        rcv.wait_recv()

    sl = strip_rows(S)
    remote_copy(sl, obuf.at[sl], x_ag_send.at[0], x_ag_recv.at[0],
                x_partner)
    for h in range(N_Y - 1):
        s = lax.rem(yy + 1 - h + 2 * N_Y, N_Y)
        sl = strip_rows(s)
        op = remote_copy(sl, obuf.at[sl],
                         y_ag_send.at[h], y_ag_recv.at[h], y_next)
        op.wait_recv()
        rl = strip_rows(lax.rem(yy - h + 2 * N_Y, N_Y))
        remote_copy(rl, obuf.at[rl],
                    x_ag_send.at[h + 1], x_ag_recv.at[h + 1], x_partner)
    for i in range(N_Y):
        rcv = pltpu.make_async_remote_copy(
            src_ref=xstage.at[pl.ds(0, STRIP)],
            dst_ref=xstage.at[pl.ds(0, STRIP)],
            send_sem=x_ag_send.at[i], recv_sem=x_ag_recv.at[i],
            device_id=(x_partner,), device_id_type=pl.DeviceIdType.MESH,
        )
        rcv.wait_recv()

    out_ref[:, :] = obuf[:, :].astype(jnp.float32)

    for op in ops:
        op.wait_send()

    @functools.partial(pl.run_scoped, sem=pltpu.SemaphoreType.REGULAR)
    def _(sem):
        for nbr in (x_partner, y_next, y_prev, *zdev):
            pl.semaphore_signal(sem, inc=1, device_id=(nbr,),
                                device_id_type=pl.DeviceIdType.MESH)
        pl.semaphore_wait(sem, 6)


def kernel(x, router_W, route_idx, expert_W):
    del router_W
    ny1 = N_Y - 1
    nz1 = N_Z - 1
    dma = pltpu.SemaphoreType.DMA
    return pl.pallas_call(
        _moe_body,
        out_shape=jax.ShapeDtypeStruct((N_TOK, H), jnp.float32),
        in_specs=[
            pl.BlockSpec(memory_space=pltpu.VMEM),
            pl.BlockSpec(memory_space=pltpu.VMEM),
            pl.BlockSpec(memory_space=pltpu.VMEM),
        ],
        out_specs=pl.BlockSpec(memory_space=pltpu.VMEM),
        scratch_shapes=[
            pltpu.VMEM((N_TOK, H), jnp.bfloat16),
            pltpu.VMEM((E_PER, D, H), jnp.bfloat16),
            pltpu.VMEM((HALF, H), jnp.bfloat16),
            pltpu.VMEM((ny1, STRIP, H), jnp.bfloat16),
            pltpu.VMEM((nz1, SUB, H), jnp.bfloat16),
            dma((N_Y,)), dma((N_Y,)),
            dma((ny1,)), dma((ny1,)),
            dma((nz1,)), dma((nz1,)),
            dma((nz1,)), dma((nz1,)),
            dma((ny1,)), dma((ny1,)),
            dma((N_Y,)), dma((N_Y,)),
        ],
        compiler_params=pltpu.CompilerParams(collective_id=0),
    )(x, route_idx.astype(jnp.int32), expert_W)
